# Initial kernel scaffold; baseline (speedup 1.0000x reference)
import jax
import jax.numpy as jnp
from jax.experimental import pallas as pl


def kernel(x, edge_attr, node_W, node_b, conv_W, conv_b, ln_g, ln_b, lin_W, lin_b, edge_index):
    raise NotImplementedError("write your pallas kernel here")


# speedup vs baseline: 8.0555x; 8.0555x over previous
import functools

import jax
import jax.numpy as jnp
from jax import lax
from jax.experimental import pallas as pl
from jax.experimental.pallas import tpu as pltpu
from jax.experimental.pallas import tpu_sc as plsc

N = 10000
E = 320000
F = 128
ODIM = 112
NLAYERS = 7

NC = 2
NS = 16
NW = NC * NS
EPW = E // NW
CHUNK = 80
NCHUNK = EPW // CHUNK
STRIPE = 640

_MESH = plsc.VectorSubcoreMesh(core_axis_name="c", subcore_axis_name="s")
_SC_PARAMS = pltpu.CompilerParams(needs_layout_passes=False)

_ROWBLK = 2000
_GRID = N // _ROWBLK


def _zero_rows(rows_ref, nrow, width):
    zero = jnp.zeros((16,), jnp.float32)

    def zr(j, carry):
        for cc in range(width // 16):
            rows_ref[j, pl.ds(16 * cc, 16)] = zero
        return carry

    lax.fori_loop(0, nrow, zr, 0)


def _stripe_info(s):
    base = s * STRIPE
    ncop = jnp.where(s == NS - 1, 5, 8).astype(jnp.int32)
    return base, ncop


def _zero_stripe(acc, rows_ref, base, ncop):
    def cp(k, carry):
        pltpu.sync_copy(rows_ref, acc.at[pl.ds(base + CHUNK * k, CHUNK)])
        return carry

    lax.fori_loop(0, ncop, cp, 0)


def _write_stripe(acc, out, c, base, ncop):
    def cp(k, carry):
        sl = pl.ds(base + CHUNK * k, CHUNK)
        pltpu.sync_copy(acc.at[sl], out.at[c, sl])
        return carry

    lax.fori_loop(0, ncop, cp, 0)


def _norm_body(srcf, dstf, ewf, dinv_h, normf, src_v, dst_v, ew_v, nrm_v, dinv_v):
    c = lax.axis_index("c")
    s = lax.axis_index("s")
    w = s * NC + c
    pltpu.sync_copy(srcf.at[w], src_v)
    pltpu.sync_copy(dstf.at[w], dst_v)
    pltpu.sync_copy(ewf.at[w], ew_v)
    pltpu.sync_copy(dinv_h, dinv_v)

    def chunk_body(b, carry):
        sl = pl.ds(b * 16, 16)
        nv = (plsc.load_gather(dinv_v, [src_v[sl]])
              * ew_v[sl]
              * plsc.load_gather(dinv_v, [dst_v[sl]]))
        nrm_v[sl] = nv
        return carry

    lax.fori_loop(0, EPW // 16, chunk_body, 0)
    pltpu.sync_copy(nrm_v, normf.at[w])


_norm_call = pl.kernel(
    _norm_body,
    out_type=jax.ShapeDtypeStruct((NW, EPW), jnp.float32),
    mesh=_MESH,
    compiler_params=_SC_PARAMS,
    scratch_types=[
        pltpu.VMEM((EPW,), jnp.int32),
        pltpu.VMEM((EPW,), jnp.int32),
        pltpu.VMEM((EPW,), jnp.float32),
        pltpu.VMEM((EPW,), jnp.float32),
        pltpu.VMEM((N,), jnp.float32),
    ],
)


def _agg_body(t_h, srcf, dstb, normf, out, src_v, dst_v, nrm_v, rows_v, acc, sem):
    c = lax.axis_index("c")
    s = lax.axis_index("s")
    w = s * NC + c
    pltpu.sync_copy(srcf.at[w], src_v)
    pltpu.sync_copy(dstb.at[w], dst_v)
    pltpu.sync_copy(normf.at[w], nrm_v)
    _zero_rows(rows_v, CHUNK, F)
    base, ncop = _stripe_info(s)
    _zero_stripe(acc, rows_v, base, ncop)
    plsc.subcore_barrier()

    def chunk_body(g, carry):
        pltpu.async_copy(
            t_h.at[src_v.at[pl.ds(g * CHUNK, CHUNK)]], rows_v, sem).wait()

        def scale(j, carry2):
            nrm = plsc.load_gather(
                nrm_v, [jnp.full((16,), g * CHUNK + j, jnp.int32)])
            for cc in range(F // 16):
                sl = pl.ds(16 * cc, 16)
                rows_v[j, sl] = rows_v[j, sl] * nrm
            return carry2

        lax.fori_loop(0, CHUNK, scale, 0)
        pltpu.sync_copy(rows_v, acc.at[dst_v.at[g]], add=True)
        return carry

    lax.fori_loop(0, NCHUNK, chunk_body, 0)
    plsc.subcore_barrier()
    _write_stripe(acc, out, c, base, ncop)


_agg_call = pl.kernel(
    _agg_body,
    out_type=jax.ShapeDtypeStruct((NC, N, F), jnp.float32),
    mesh=_MESH,
    compiler_params=_SC_PARAMS,
    scratch_types=[
        pltpu.VMEM((EPW,), jnp.int32),
        pltpu.VMEM((NCHUNK, CHUNK), jnp.int32),
        pltpu.VMEM((EPW,), jnp.float32),
        pltpu.VMEM((CHUNK, F), jnp.float32),
        pltpu.VMEM_SHARED((N, F), jnp.float32),
        pltpu.SemaphoreType.DMA,
    ],
)


def _prep_body(x_ref, nw_ref, nb_ref, degp_ref, t_ref, dinv_ref):
    deg = degp_ref[0, :, 0] + degp_ref[1, :, 0] + 1.0
    dinv_ref[0, 0] = lax.rsqrt(deg)
    t_ref[...] = (jnp.dot(x_ref[...], nw_ref[...],
                          preferred_element_type=jnp.float32) + nb_ref[...])


_prep_call = pl.pallas_call(
    _prep_body,
    grid=(_GRID,),
    in_specs=[
        pl.BlockSpec((_ROWBLK, F), lambda i: (i, 0)),
        pl.BlockSpec((F, F), lambda i: (0, 0)),
        pl.BlockSpec((F,), lambda i: (0,)),
        pl.BlockSpec((NC, _ROWBLK, F), lambda i: (0, i, 0)),
    ],
    out_specs=[
        pl.BlockSpec((_ROWBLK, F), lambda i: (i, 0)),
        pl.BlockSpec((1, 1, _ROWBLK), lambda i: (i, 0, 0)),
    ],
    out_shape=[
        jax.ShapeDtypeStruct((N, F), jnp.float32),
        jax.ShapeDtypeStruct((_GRID, 1, _ROWBLK), jnp.float32),
    ],
)


def _layer_body(first, p_ref, t_ref, dinv_ref, h_ref, w_ref, b_ref, g_ref,
                bb_ref, ho_ref, to_ref):
    dinv = dinv_ref[0, 0]
    agg = p_ref[0] + p_ref[1] + t_ref[...] * (dinv * dinv)[:, None]
    conv = jnp.dot(agg, w_ref[...], preferred_element_type=jnp.float32) + b_ref[...]
    h = conv if first else h_ref[...] + conv
    ho_ref[...] = h
    mu = jnp.mean(h, axis=-1, keepdims=True)
    var = jnp.mean((h - mu) ** 2, axis=-1, keepdims=True)
    to_ref[...] = jnp.maximum(
        (h - mu) * lax.rsqrt(var + 1e-5) * g_ref[...] + bb_ref[...], 0.0)


def _make_layer_call(first):
    return pl.pallas_call(
        functools.partial(_layer_body, first),
        grid=(_GRID,),
        in_specs=[
            pl.BlockSpec((NC, _ROWBLK, F), lambda i: (0, i, 0)),
            pl.BlockSpec((_ROWBLK, F), lambda i: (i, 0)),
            pl.BlockSpec((1, 1, _ROWBLK), lambda i: (i, 0, 0)),
            pl.BlockSpec((_ROWBLK, F), lambda i: (i, 0)),
            pl.BlockSpec((F, F), lambda i: (0, 0)),
            pl.BlockSpec((F,), lambda i: (0,)),
            pl.BlockSpec((F,), lambda i: (0,)),
            pl.BlockSpec((F,), lambda i: (0,)),
        ],
        out_specs=[
            pl.BlockSpec((_ROWBLK, F), lambda i: (i, 0)),
            pl.BlockSpec((_ROWBLK, F), lambda i: (i, 0)),
        ],
        out_shape=[
            jax.ShapeDtypeStruct((N, F), jnp.float32),
            jax.ShapeDtypeStruct((N, F), jnp.float32),
        ],
    )


_layer_first = _make_layer_call(True)
_layer_rest = _make_layer_call(False)


def _final_body(t_ref, lw_ref, lb_ref, o_ref):
    o_ref[...] = (jnp.dot(t_ref[...], lw_ref[...],
                          preferred_element_type=jnp.float32) + lb_ref[...])


_final_call = pl.pallas_call(
    _final_body,
    grid=(_GRID,),
    in_specs=[
        pl.BlockSpec((_ROWBLK, F), lambda i: (i, 0)),
        pl.BlockSpec((F, ODIM), lambda i: (0, 0)),
        pl.BlockSpec((ODIM,), lambda i: (0,)),
    ],
    out_specs=pl.BlockSpec((_ROWBLK, ODIM), lambda i: (i, 0)),
    out_shape=jax.ShapeDtypeStruct((N, ODIM), jnp.float32),
)


def kernel(x, edge_attr, node_W, node_b, conv_W, conv_b, ln_g, ln_b, lin_W,
           lin_b, edge_index):
    srcf = edge_index[0].reshape(NW, EPW)
    dstf = edge_index[1].reshape(NW, EPW)
    dstb = edge_index[1].reshape(NW, NCHUNK, CHUNK)
    ewf = edge_attr.reshape(NW, EPW)

    ones = jnp.ones((N, F), jnp.float32)
    degp = _agg_call(ones, srcf, dstb, ewf)
    t, dinv = _prep_call(x, node_W, node_b, degp)
    normb = _norm_call(srcf, dstf, ewf, dinv.reshape(N))

    h = t
    for i in range(NLAYERS):
        p = _agg_call(t, srcf, dstb, normb)
        gi = ln_g[i + 1] if i + 1 < NLAYERS else ln_g[0]
        bi = ln_b[i + 1] if i + 1 < NLAYERS else ln_b[0]
        call = _layer_first if i == 0 else _layer_rest
        h, t = call(p, t, dinv, h, conv_W[i], conv_b[i], gi, bi)
    return _final_call(t, lin_W, lin_b)


# speedup vs baseline: 14.4249x; 1.7907x over previous
import functools

import jax
import jax.numpy as jnp
from jax import lax
from jax.experimental import pallas as pl
from jax.experimental.pallas import tpu as pltpu
from jax.experimental.pallas import tpu_sc as plsc

N = 10000
E = 320000
F = 128
ODIM = 112
NLAYERS = 7

NC = 2
NS = 16
NW = NC * NS
EPW = E // NW
CHUNK = 125
NCHUNK = EPW // CHUNK
NPAIR = NCHUNK // 2
STRIPE = 640

_MESH = plsc.VectorSubcoreMesh(core_axis_name="c", subcore_axis_name="s")
_SC_PARAMS = pltpu.CompilerParams(needs_layout_passes=False)

_ROWBLK = 2000
_GRID = N // _ROWBLK


def _zero_rows(rows_ref, nrow, width):
    zero = jnp.zeros((16,), jnp.float32)

    def zr(j, carry):
        for cc in range(width // 16):
            rows_ref[j, pl.ds(16 * cc, 16)] = zero
        return carry

    lax.fori_loop(0, nrow, zr, 0)


def _stripe_info(s):
    base = s * STRIPE
    ncop = jnp.where(s == NS - 1, 5, 8).astype(jnp.int32)
    return base, ncop


def _zero_stripe(acc, rows_ref, base, ncop):
    def cp(k, carry):
        pltpu.sync_copy(rows_ref.at[pl.ds(0, 80)],
                        acc.at[pl.ds(base + 80 * k, 80)])
        return carry

    lax.fori_loop(0, ncop, cp, 0)


def _write_stripe(acc, out, c, base, ncop):
    def cp(k, carry):
        sl = pl.ds(base + 80 * k, 80)
        pltpu.sync_copy(acc.at[sl], out.at[c, sl])
        return carry

    lax.fori_loop(0, ncop, cp, 0)


def _norm_body(srcf, dstf, ewf, dinv_h, normf, src_v, dst_v, ew_v, nrm_v, dinv_v):
    c = lax.axis_index("c")
    s = lax.axis_index("s")
    w = s * NC + c
    pltpu.sync_copy(srcf.at[w], src_v)
    pltpu.sync_copy(dstf.at[w], dst_v)
    pltpu.sync_copy(ewf.at[w], ew_v)
    pltpu.sync_copy(dinv_h, dinv_v)

    def chunk_body(b, carry):
        sl = pl.ds(b * 16, 16)
        nv = (plsc.load_gather(dinv_v, [src_v[sl]])
              * ew_v[sl]
              * plsc.load_gather(dinv_v, [dst_v[sl]]))
        nrm_v[sl] = nv
        return carry

    lax.fori_loop(0, EPW // 16, chunk_body, 0)
    pltpu.sync_copy(nrm_v, normf.at[w])


_norm_call = pl.kernel(
    _norm_body,
    out_type=jax.ShapeDtypeStruct((NW, EPW), jnp.float32),
    mesh=_MESH,
    compiler_params=_SC_PARAMS,
    scratch_types=[
        pltpu.VMEM((EPW,), jnp.int32),
        pltpu.VMEM((EPW,), jnp.int32),
        pltpu.VMEM((EPW,), jnp.float32),
        pltpu.VMEM((EPW,), jnp.float32),
        pltpu.VMEM((N,), jnp.float32),
    ],
)


def _agg_body(t_h, srcb, dstb, normb, out, dst_v, src_a, src_b, nrm_a, nrm_b,
              rows_a, rows_b, acc, sem_ga, sem_gb, sem_ma, sem_mb):
    c = lax.axis_index("c")
    s = lax.axis_index("s")
    w = s * NC + c
    pltpu.sync_copy(dstb.at[w], dst_v)
    _zero_rows(rows_a, 80, F)
    base, ncop = _stripe_info(s)
    _zero_stripe(acc, rows_a, base, ncop)

    def stage_meta_sync(g, src_buf, nrm_buf):
        pltpu.sync_copy(srcb.at[w, g], src_buf)
        pltpu.sync_copy(normb.at[w, g], nrm_buf)

    def stage_meta(g, src_buf, nrm_buf, sem_m):
        pltpu.async_copy(srcb.at[w, g], src_buf, sem_m)
        pltpu.async_copy(normb.at[w, g], nrm_buf, sem_m)

    def wait_meta(g, src_buf, nrm_buf, sem_m):
        pltpu.make_async_copy(srcb.at[w, g], src_buf, sem_m).wait()
        pltpu.make_async_copy(normb.at[w, g], nrm_buf, sem_m).wait()

    def gather(buf, src_buf, sem_g):
        pltpu.async_copy(t_h.at[src_buf], buf, sem_g)

    def wait_gather(buf, src_buf, sem_g):
        pltpu.make_async_copy(t_h.at[src_buf], buf, sem_g).wait()

    def scale(buf, nrm_buf):
        def body(jj, carry2):
            for u in range(5):
                j = jj * 5 + u
                nrm = plsc.load_gather(
                    nrm_buf, [jnp.full((16,), j, jnp.int32)])
                for cc in range(F // 16):
                    sl = pl.ds(16 * cc, 16)
                    buf[j, sl] = buf[j, sl] * nrm
            return carry2

        lax.fori_loop(0, CHUNK // 5, body, 0)

    def scatter(g, buf):
        pltpu.sync_copy(buf, acc.at[dst_v.at[g]], add=True)

    stage_meta_sync(0, src_a, nrm_a)
    stage_meta_sync(1, src_b, nrm_b)
    gather(rows_a, src_a, sem_ga)
    gather(rows_b, src_b, sem_gb)
    plsc.subcore_barrier()

    def pair_body(k, carry):
        a = 2 * k
        more = k < NPAIR - 1
        wait_gather(rows_a, src_a, sem_ga)
        scale(rows_a, nrm_a)

        @pl.when(more)
        def _():
            stage_meta(a + 2, src_a, nrm_a, sem_ma)

        scatter(a, rows_a)

        @pl.when(more)
        def _():
            wait_meta(a + 2, src_a, nrm_a, sem_ma)
            gather(rows_a, src_a, sem_ga)

        wait_gather(rows_b, src_b, sem_gb)
        scale(rows_b, nrm_b)

        @pl.when(more)
        def _():
            stage_meta(a + 3, src_b, nrm_b, sem_mb)

        scatter(a + 1, rows_b)

        @pl.when(more)
        def _():
            wait_meta(a + 3, src_b, nrm_b, sem_mb)
            gather(rows_b, src_b, sem_gb)

        return carry

    lax.fori_loop(0, NPAIR, pair_body, 0)
    plsc.subcore_barrier()
    _write_stripe(acc, out, c, base, ncop)


_agg_call = pl.kernel(
    _agg_body,
    out_type=jax.ShapeDtypeStruct((NC, N, F), jnp.float32),
    mesh=_MESH,
    compiler_params=_SC_PARAMS,
    scratch_types=[
        pltpu.VMEM((NCHUNK, CHUNK), jnp.int32),
        pltpu.VMEM((CHUNK,), jnp.int32),
        pltpu.VMEM((CHUNK,), jnp.int32),
        pltpu.VMEM((CHUNK,), jnp.float32),
        pltpu.VMEM((CHUNK,), jnp.float32),
        pltpu.VMEM((CHUNK, F), jnp.float32),
        pltpu.VMEM((CHUNK, F), jnp.float32),
        pltpu.VMEM_SHARED((N, F), jnp.float32),
        pltpu.SemaphoreType.DMA,
        pltpu.SemaphoreType.DMA,
        pltpu.SemaphoreType.DMA,
        pltpu.SemaphoreType.DMA,
    ],
)


def _prep_body(x_ref, nw_ref, nb_ref, degp_ref, t_ref, dinv_ref):
    deg = degp_ref[0, :, 0] + degp_ref[1, :, 0] + 1.0
    dinv_ref[0, 0] = lax.rsqrt(deg)
    t_ref[...] = (jnp.dot(x_ref[...], nw_ref[...],
                          preferred_element_type=jnp.float32) + nb_ref[...])


_prep_call = pl.pallas_call(
    _prep_body,
    grid=(_GRID,),
    in_specs=[
        pl.BlockSpec((_ROWBLK, F), lambda i: (i, 0)),
        pl.BlockSpec((F, F), lambda i: (0, 0)),
        pl.BlockSpec((F,), lambda i: (0,)),
        pl.BlockSpec((NC, _ROWBLK, F), lambda i: (0, i, 0)),
    ],
    out_specs=[
        pl.BlockSpec((_ROWBLK, F), lambda i: (i, 0)),
        pl.BlockSpec((1, 1, _ROWBLK), lambda i: (i, 0, 0)),
    ],
    out_shape=[
        jax.ShapeDtypeStruct((N, F), jnp.float32),
        jax.ShapeDtypeStruct((_GRID, 1, _ROWBLK), jnp.float32),
    ],
)


def _layer_body(first, p_ref, t_ref, dinv_ref, h_ref, w_ref, b_ref, g_ref,
                bb_ref, ho_ref, to_ref):
    dinv = dinv_ref[0, 0]
    agg = p_ref[0] + p_ref[1] + t_ref[...] * (dinv * dinv)[:, None]
    conv = jnp.dot(agg, w_ref[...], preferred_element_type=jnp.float32) + b_ref[...]
    h = conv if first else h_ref[...] + conv
    ho_ref[...] = h
    mu = jnp.mean(h, axis=-1, keepdims=True)
    var = jnp.mean((h - mu) ** 2, axis=-1, keepdims=True)
    to_ref[...] = jnp.maximum(
        (h - mu) * lax.rsqrt(var + 1e-5) * g_ref[...] + bb_ref[...], 0.0)


def _make_layer_call(first):
    return pl.pallas_call(
        functools.partial(_layer_body, first),
        grid=(_GRID,),
        in_specs=[
            pl.BlockSpec((NC, _ROWBLK, F), lambda i: (0, i, 0)),
            pl.BlockSpec((_ROWBLK, F), lambda i: (i, 0)),
            pl.BlockSpec((1, 1, _ROWBLK), lambda i: (i, 0, 0)),
            pl.BlockSpec((_ROWBLK, F), lambda i: (i, 0)),
            pl.BlockSpec((F, F), lambda i: (0, 0)),
            pl.BlockSpec((F,), lambda i: (0,)),
            pl.BlockSpec((F,), lambda i: (0,)),
            pl.BlockSpec((F,), lambda i: (0,)),
        ],
        out_specs=[
            pl.BlockSpec((_ROWBLK, F), lambda i: (i, 0)),
            pl.BlockSpec((_ROWBLK, F), lambda i: (i, 0)),
        ],
        out_shape=[
            jax.ShapeDtypeStruct((N, F), jnp.float32),
            jax.ShapeDtypeStruct((N, F), jnp.float32),
        ],
    )


_layer_first = _make_layer_call(True)
_layer_rest = _make_layer_call(False)


def _final_body(t_ref, lw_ref, lb_ref, o_ref):
    o_ref[...] = (jnp.dot(t_ref[...], lw_ref[...],
                          preferred_element_type=jnp.float32) + lb_ref[...])


_final_call = pl.pallas_call(
    _final_body,
    grid=(_GRID,),
    in_specs=[
        pl.BlockSpec((_ROWBLK, F), lambda i: (i, 0)),
        pl.BlockSpec((F, ODIM), lambda i: (0, 0)),
        pl.BlockSpec((ODIM,), lambda i: (0,)),
    ],
    out_specs=pl.BlockSpec((_ROWBLK, ODIM), lambda i: (i, 0)),
    out_shape=jax.ShapeDtypeStruct((N, ODIM), jnp.float32),
)


def kernel(x, edge_attr, node_W, node_b, conv_W, conv_b, ln_g, ln_b, lin_W,
           lin_b, edge_index):
    srcf = edge_index[0].reshape(NW, EPW)
    dstf = edge_index[1].reshape(NW, EPW)
    srcb = edge_index[0].reshape(NW, NCHUNK, CHUNK)
    dstb = edge_index[1].reshape(NW, NCHUNK, CHUNK)
    ewf = edge_attr.reshape(NW, EPW)
    ewb = edge_attr.reshape(NW, NCHUNK, CHUNK)

    ones = jnp.ones((N, F), jnp.float32)
    degp = _agg_call(ones, srcb, dstb, ewb)
    t, dinv = _prep_call(x, node_W, node_b, degp)
    normb = _norm_call(srcf, dstf, ewf, dinv.reshape(N)).reshape(
        NW, NCHUNK, CHUNK)

    h = t
    for i in range(NLAYERS):
        p = _agg_call(t, srcb, dstb, normb)
        gi = ln_g[i + 1] if i + 1 < NLAYERS else ln_g[0]
        bi = ln_b[i + 1] if i + 1 < NLAYERS else ln_b[0]
        call = _layer_first if i == 0 else _layer_rest
        h, t = call(p, t, dinv, h, conv_W[i], conv_b[i], gi, bi)
    return _final_call(t, lin_W, lin_b)


# speedup vs baseline: 15.4564x; 1.0715x over previous
import functools

import jax
import jax.numpy as jnp
from jax import lax
from jax.experimental import pallas as pl
from jax.experimental.pallas import tpu as pltpu
from jax.experimental.pallas import tpu_sc as plsc

N = 10000
E = 320000
F = 128
ODIM = 112
NLAYERS = 7

NC = 2
NS = 16
NW = NC * NS
EPW = E // NW
CHUNK = 125
NCHUNK = EPW // CHUNK
NPAIR = NCHUNK // 2
STRIPE = 640

_MESH = plsc.VectorSubcoreMesh(core_axis_name="c", subcore_axis_name="s")
_SC_PARAMS = pltpu.CompilerParams(needs_layout_passes=False)

_ROWBLK = 2000
_GRID = N // _ROWBLK


def _zero_rows(rows_ref, nrow, width):
    zero = jnp.zeros((16,), jnp.float32)

    def zr(j, carry):
        for cc in range(width // 16):
            rows_ref[j, pl.ds(16 * cc, 16)] = zero
        return carry

    lax.fori_loop(0, nrow, zr, 0)


def _stripe_info(s):
    base = s * STRIPE
    ncop = jnp.where(s == NS - 1, 5, 8).astype(jnp.int32)
    return base, ncop


def _zero_stripe(acc, rows_ref, base, ncop):
    def cp(k, carry):
        pltpu.sync_copy(rows_ref.at[pl.ds(0, 80)],
                        acc.at[pl.ds(base + 80 * k, 80)])
        return carry

    lax.fori_loop(0, ncop, cp, 0)


def _write_stripe(acc, out, c, base, ncop):
    def cp(k, carry):
        sl = pl.ds(base + 80 * k, 80)
        pltpu.sync_copy(acc.at[sl], out.at[c, sl])
        return carry

    lax.fori_loop(0, ncop, cp, 0)


def _deg_body(dstb, ewb, out, dst_v, ew_a, ew_b, rows_a, rows_b, acc,
              sem_ma, sem_mb):
    c = lax.axis_index("c")
    s = lax.axis_index("s")
    w = s * NC + c
    pltpu.sync_copy(dstb.at[w], dst_v)
    _zero_rows(rows_a, 80, 16)
    base, ncop = _stripe_info(s)
    _zero_stripe(acc, rows_a, base, ncop)
    pltpu.sync_copy(ewb.at[w, 0], ew_a)
    pltpu.sync_copy(ewb.at[w, 1], ew_b)
    plsc.subcore_barrier()

    def build(buf, ew_buf):
        def body(jj, carry2):
            for u in range(5):
                j = jj * 5 + u
                buf[j, :] = plsc.load_gather(
                    ew_buf, [jnp.full((16,), j, jnp.int32)])
            return carry2

        lax.fori_loop(0, CHUNK // 5, body, 0)

    def pair_body(k, carry):
        a = 2 * k
        more = k < NPAIR - 1
        build(rows_a, ew_a)

        @pl.when(more)
        def _():
            pltpu.async_copy(ewb.at[w, a + 2], ew_a, sem_ma)

        pltpu.sync_copy(rows_a, acc.at[dst_v.at[a]], add=True)
        build(rows_b, ew_b)

        @pl.when(more)
        def _():
            pltpu.async_copy(ewb.at[w, a + 3], ew_b, sem_mb)

        pltpu.sync_copy(rows_b, acc.at[dst_v.at[a + 1]], add=True)

        @pl.when(more)
        def _():
            pltpu.make_async_copy(ewb.at[w, a + 2], ew_a, sem_ma).wait()
            pltpu.make_async_copy(ewb.at[w, a + 3], ew_b, sem_mb).wait()

        return carry

    lax.fori_loop(0, NPAIR, pair_body, 0)
    plsc.subcore_barrier()
    _write_stripe(acc, out, c, base, ncop)


_deg_call = pl.kernel(
    _deg_body,
    out_type=jax.ShapeDtypeStruct((NC, N, 16), jnp.float32),
    mesh=_MESH,
    compiler_params=_SC_PARAMS,
    scratch_types=[
        pltpu.VMEM((NCHUNK, CHUNK), jnp.int32),
        pltpu.VMEM((CHUNK,), jnp.float32),
        pltpu.VMEM((CHUNK,), jnp.float32),
        pltpu.VMEM((CHUNK, 16), jnp.float32),
        pltpu.VMEM((CHUNK, 16), jnp.float32),
        pltpu.VMEM_SHARED((N, 16), jnp.float32),
        pltpu.SemaphoreType.DMA,
        pltpu.SemaphoreType.DMA,
    ],
)


def _norm_body(srcf, dstf, ewf, dinv_h, normf, src_v, dst_v, ew_v, nrm_v, dinv_v):
    c = lax.axis_index("c")
    s = lax.axis_index("s")
    w = s * NC + c
    pltpu.sync_copy(srcf.at[w], src_v)
    pltpu.sync_copy(dstf.at[w], dst_v)
    pltpu.sync_copy(ewf.at[w], ew_v)
    pltpu.sync_copy(dinv_h, dinv_v)

    def chunk_body(b, carry):
        sl = pl.ds(b * 16, 16)
        nv = (plsc.load_gather(dinv_v, [src_v[sl]])
              * ew_v[sl]
              * plsc.load_gather(dinv_v, [dst_v[sl]]))
        nrm_v[sl] = nv
        return carry

    lax.fori_loop(0, EPW // 16, chunk_body, 0)
    pltpu.sync_copy(nrm_v, normf.at[w])


_norm_call = pl.kernel(
    _norm_body,
    out_type=jax.ShapeDtypeStruct((NW, EPW), jnp.float32),
    mesh=_MESH,
    compiler_params=_SC_PARAMS,
    scratch_types=[
        pltpu.VMEM((EPW,), jnp.int32),
        pltpu.VMEM((EPW,), jnp.int32),
        pltpu.VMEM((EPW,), jnp.float32),
        pltpu.VMEM((EPW,), jnp.float32),
        pltpu.VMEM((N,), jnp.float32),
    ],
)


def _agg_body(t_h, srcb, dstb, normb, out, dst_v, src_a, src_b, nrm_a, nrm_b,
              rows_a, rows_b, acc, sem_ga, sem_gb, sem_ma, sem_mb):
    c = lax.axis_index("c")
    s = lax.axis_index("s")
    w = s * NC + c
    pltpu.sync_copy(dstb.at[w], dst_v)
    _zero_rows(rows_a, 80, F)
    base, ncop = _stripe_info(s)
    _zero_stripe(acc, rows_a, base, ncop)

    def stage_meta_sync(g, src_buf, nrm_buf):
        pltpu.sync_copy(srcb.at[w, g], src_buf)
        pltpu.sync_copy(normb.at[w, g], nrm_buf)

    def stage_meta(g, src_buf, nrm_buf, sem_m):
        pltpu.async_copy(srcb.at[w, g], src_buf, sem_m)
        pltpu.async_copy(normb.at[w, g], nrm_buf, sem_m)

    def wait_meta(g, src_buf, nrm_buf, sem_m):
        pltpu.make_async_copy(srcb.at[w, g], src_buf, sem_m).wait()
        pltpu.make_async_copy(normb.at[w, g], nrm_buf, sem_m).wait()

    def gather(buf, src_buf, sem_g):
        pltpu.async_copy(t_h.at[src_buf], buf, sem_g)

    def wait_gather(buf, src_buf, sem_g):
        pltpu.make_async_copy(t_h.at[src_buf], buf, sem_g).wait()

    def scale(buf, nrm_buf):
        def body(jj, carry2):
            for u in range(5):
                j = jj * 5 + u
                nrm = plsc.load_gather(
                    nrm_buf, [jnp.full((16,), j, jnp.int32)])
                for cc in range(F // 16):
                    sl = pl.ds(16 * cc, 16)
                    buf[j, sl] = buf[j, sl] * nrm
            return carry2

        lax.fori_loop(0, CHUNK // 5, body, 0)

    def scatter(g, buf):
        pltpu.sync_copy(buf, acc.at[dst_v.at[g]], add=True)

    stage_meta_sync(0, src_a, nrm_a)
    stage_meta_sync(1, src_b, nrm_b)
    gather(rows_a, src_a, sem_ga)
    gather(rows_b, src_b, sem_gb)
    plsc.subcore_barrier()

    def pair_body(k, carry):
        a = 2 * k
        more = k < NPAIR - 1
        wait_gather(rows_a, src_a, sem_ga)
        scale(rows_a, nrm_a)

        @pl.when(more)
        def _():
            stage_meta(a + 2, src_a, nrm_a, sem_ma)

        scatter(a, rows_a)

        @pl.when(more)
        def _():
            wait_meta(a + 2, src_a, nrm_a, sem_ma)
            gather(rows_a, src_a, sem_ga)

        wait_gather(rows_b, src_b, sem_gb)
        scale(rows_b, nrm_b)

        @pl.when(more)
        def _():
            stage_meta(a + 3, src_b, nrm_b, sem_mb)

        scatter(a + 1, rows_b)

        @pl.when(more)
        def _():
            wait_meta(a + 3, src_b, nrm_b, sem_mb)
            gather(rows_b, src_b, sem_gb)

        return carry

    lax.fori_loop(0, NPAIR, pair_body, 0)
    plsc.subcore_barrier()
    _write_stripe(acc, out, c, base, ncop)


_agg_call = pl.kernel(
    _agg_body,
    out_type=jax.ShapeDtypeStruct((NC, N, F), jnp.float32),
    mesh=_MESH,
    compiler_params=_SC_PARAMS,
    scratch_types=[
        pltpu.VMEM((NCHUNK, CHUNK), jnp.int32),
        pltpu.VMEM((CHUNK,), jnp.int32),
        pltpu.VMEM((CHUNK,), jnp.int32),
        pltpu.VMEM((CHUNK,), jnp.float32),
        pltpu.VMEM((CHUNK,), jnp.float32),
        pltpu.VMEM((CHUNK, F), jnp.float32),
        pltpu.VMEM((CHUNK, F), jnp.float32),
        pltpu.VMEM_SHARED((N, F), jnp.float32),
        pltpu.SemaphoreType.DMA,
        pltpu.SemaphoreType.DMA,
        pltpu.SemaphoreType.DMA,
        pltpu.SemaphoreType.DMA,
    ],
)


def _prep_body(x_ref, nw_ref, nb_ref, degp_ref, t_ref, dinv_ref):
    deg = degp_ref[0, :, 0] + degp_ref[1, :, 0] + 1.0
    dinv_ref[0, 0] = lax.rsqrt(deg)
    t_ref[...] = (jnp.dot(x_ref[...], nw_ref[...],
                          preferred_element_type=jnp.float32) + nb_ref[...])


_prep_call = pl.pallas_call(
    _prep_body,
    grid=(_GRID,),
    in_specs=[
        pl.BlockSpec((_ROWBLK, F), lambda i: (i, 0)),
        pl.BlockSpec((F, F), lambda i: (0, 0)),
        pl.BlockSpec((F,), lambda i: (0,)),
        pl.BlockSpec((NC, _ROWBLK, 16), lambda i: (0, i, 0)),
    ],
    out_specs=[
        pl.BlockSpec((_ROWBLK, F), lambda i: (i, 0)),
        pl.BlockSpec((1, 1, _ROWBLK), lambda i: (i, 0, 0)),
    ],
    out_shape=[
        jax.ShapeDtypeStruct((N, F), jnp.float32),
        jax.ShapeDtypeStruct((_GRID, 1, _ROWBLK), jnp.float32),
    ],
)


def _layer_body(first, p_ref, t_ref, dinv_ref, h_ref, w_ref, b_ref, g_ref,
                bb_ref, ho_ref, to_ref, *lin_refs):
    dinv = dinv_ref[0, 0]
    agg = p_ref[0] + p_ref[1] + t_ref[...] * (dinv * dinv)[:, None]
    conv = jnp.dot(agg, w_ref[...], preferred_element_type=jnp.float32) + b_ref[...]
    h = conv if first else h_ref[...] + conv
    if ho_ref is not None:
        ho_ref[...] = h
    mu = jnp.mean(h, axis=-1, keepdims=True)
    var = jnp.mean((h - mu) ** 2, axis=-1, keepdims=True)
    t_next = jnp.maximum(
        (h - mu) * lax.rsqrt(var + 1e-5) * g_ref[...] + bb_ref[...], 0.0)
    if lin_refs:
        lw_ref, lb_ref = lin_refs
        t_next = (jnp.dot(t_next, lw_ref[...],
                          preferred_element_type=jnp.float32) + lb_ref[...])
    to_ref[...] = t_next


def _make_layer_call(first, last):
    def body(*refs):
        if last:
            (p, t, dinv, h, w, b, g, bb, lw, lb, to) = refs
            _layer_body(first, p, t, dinv, h, w, b, g, bb, None, to, lw, lb)
        else:
            (p, t, dinv, h, w, b, g, bb, ho, to) = refs
            _layer_body(first, p, t, dinv, h, w, b, g, bb, ho, to)

    odim = ODIM if last else F
    in_specs = [
        pl.BlockSpec((NC, _ROWBLK, F), lambda i: (0, i, 0)),
        pl.BlockSpec((_ROWBLK, F), lambda i: (i, 0)),
        pl.BlockSpec((1, 1, _ROWBLK), lambda i: (i, 0, 0)),
        pl.BlockSpec((_ROWBLK, F), lambda i: (i, 0)),
        pl.BlockSpec((F, F), lambda i: (0, 0)),
        pl.BlockSpec((F,), lambda i: (0,)),
        pl.BlockSpec((F,), lambda i: (0,)),
        pl.BlockSpec((F,), lambda i: (0,)),
    ]
    if last:
        in_specs += [
            pl.BlockSpec((F, ODIM), lambda i: (0, 0)),
            pl.BlockSpec((ODIM,), lambda i: (0,)),
        ]
        out_specs = pl.BlockSpec((_ROWBLK, ODIM), lambda i: (i, 0))
        out_shape = jax.ShapeDtypeStruct((N, ODIM), jnp.float32)
    else:
        out_specs = [
            pl.BlockSpec((_ROWBLK, F), lambda i: (i, 0)),
            pl.BlockSpec((_ROWBLK, F), lambda i: (i, 0)),
        ]
        out_shape = [
            jax.ShapeDtypeStruct((N, F), jnp.float32),
            jax.ShapeDtypeStruct((N, F), jnp.float32),
        ]
    return pl.pallas_call(
        body,
        grid=(_GRID,),
        in_specs=in_specs,
        out_specs=out_specs,
        out_shape=out_shape,
    )


_layer_first = _make_layer_call(True, False)
_layer_rest = _make_layer_call(False, False)
_layer_last = _make_layer_call(False, True)


def kernel(x, edge_attr, node_W, node_b, conv_W, conv_b, ln_g, ln_b, lin_W,
           lin_b, edge_index):
    srcf = edge_index[0].reshape(NW, EPW)
    dstf = edge_index[1].reshape(NW, EPW)
    srcb = edge_index[0].reshape(NW, NCHUNK, CHUNK)
    dstb = edge_index[1].reshape(NW, NCHUNK, CHUNK)
    ewf = edge_attr.reshape(NW, EPW)
    ewb = edge_attr.reshape(NW, NCHUNK, CHUNK)

    degp = _deg_call(dstb, ewb)
    t, dinv = _prep_call(x, node_W, node_b, degp)
    normb = _norm_call(srcf, dstf, ewf, dinv.reshape(N)).reshape(
        NW, NCHUNK, CHUNK)

    h = t
    for i in range(NLAYERS):
        p = _agg_call(t, srcb, dstb, normb)
        gi = ln_g[i + 1] if i + 1 < NLAYERS else ln_g[0]
        bi = ln_b[i + 1] if i + 1 < NLAYERS else ln_b[0]
        if i == 0:
            h, t = _layer_first(p, t, dinv, h, conv_W[i], conv_b[i], gi, bi)
        elif i < NLAYERS - 1:
            h, t = _layer_rest(p, t, dinv, h, conv_W[i], conv_b[i], gi, bi)
        else:
            return _layer_last(p, t, dinv, h, conv_W[i], conv_b[i], gi, bi,
                               lin_W, lin_b)


# speedup vs baseline: 15.7509x; 1.0191x over previous
import functools

import jax
import jax.numpy as jnp
from jax import lax
from jax.experimental import pallas as pl
from jax.experimental.pallas import tpu as pltpu
from jax.experimental.pallas import tpu_sc as plsc

N = 10000
E = 320000
F = 128
ODIM = 112
NLAYERS = 7

NC = 2
NS = 16
NW = NC * NS
EPW = E // NW
CHUNK = 125
NCHUNK = EPW // CHUNK
NPAIR = NCHUNK // 2
ACH = 50
ANCH = EPW // ACH
NTRI = (ANCH - 2) // 3
STRIPE = 640

_MESH = plsc.VectorSubcoreMesh(core_axis_name="c", subcore_axis_name="s")
_SC_PARAMS = pltpu.CompilerParams(needs_layout_passes=False)

_ROWBLK = 2000
_GRID = N // _ROWBLK


def _zero_rows(rows_ref, nrow, width):
    zero = jnp.zeros((16,), jnp.float32)

    def zr(j, carry):
        for cc in range(width // 16):
            rows_ref[j, pl.ds(16 * cc, 16)] = zero
        return carry

    lax.fori_loop(0, nrow, zr, 0)


def _stripe_info(s):
    base = s * STRIPE
    ncop = jnp.where(s == NS - 1, 10, 16).astype(jnp.int32)
    return base, ncop


def _zero_stripe(acc, rows_ref, base, ncop):
    def cp(k, carry):
        pltpu.sync_copy(rows_ref.at[pl.ds(0, 40)],
                        acc.at[pl.ds(base + 40 * k, 40)])
        return carry

    lax.fori_loop(0, ncop, cp, 0)


def _write_stripe(acc, out, c, base, ncop):
    def cp(k, carry):
        sl = pl.ds(base + 40 * k, 40)
        pltpu.sync_copy(acc.at[sl], out.at[c, sl])
        return carry

    lax.fori_loop(0, ncop, cp, 0)


def _deg_body(dstb, ewb, out, dst_v, ew_a, ew_b, rows_a, rows_b, acc,
              sem_ma, sem_mb):
    c = lax.axis_index("c")
    s = lax.axis_index("s")
    w = s * NC + c
    pltpu.sync_copy(dstb.at[w], dst_v)
    _zero_rows(rows_a, 40, 16)
    base, ncop = _stripe_info(s)
    _zero_stripe(acc, rows_a, base, ncop)
    pltpu.sync_copy(ewb.at[w, 0], ew_a)
    pltpu.sync_copy(ewb.at[w, 1], ew_b)
    plsc.subcore_barrier()

    def build(buf, ew_buf):
        def body(jj, carry2):
            for u in range(5):
                j = jj * 5 + u
                buf[j, :] = plsc.load_gather(
                    ew_buf, [jnp.full((16,), j, jnp.int32)])
            return carry2

        lax.fori_loop(0, CHUNK // 5, body, 0)

    def pair_body(k, carry):
        a = 2 * k
        more = k < NPAIR - 1
        build(rows_a, ew_a)

        @pl.when(more)
        def _():
            pltpu.async_copy(ewb.at[w, a + 2], ew_a, sem_ma)

        pltpu.sync_copy(rows_a, acc.at[dst_v.at[a]], add=True)
        build(rows_b, ew_b)

        @pl.when(more)
        def _():
            pltpu.async_copy(ewb.at[w, a + 3], ew_b, sem_mb)

        pltpu.sync_copy(rows_b, acc.at[dst_v.at[a + 1]], add=True)

        @pl.when(more)
        def _():
            pltpu.make_async_copy(ewb.at[w, a + 2], ew_a, sem_ma).wait()
            pltpu.make_async_copy(ewb.at[w, a + 3], ew_b, sem_mb).wait()

        return carry

    lax.fori_loop(0, NPAIR, pair_body, 0)
    plsc.subcore_barrier()
    _write_stripe(acc, out, c, base, ncop)


_deg_call = pl.kernel(
    _deg_body,
    out_type=jax.ShapeDtypeStruct((NC, N, 16), jnp.float32),
    mesh=_MESH,
    compiler_params=_SC_PARAMS,
    scratch_types=[
        pltpu.VMEM((NCHUNK, CHUNK), jnp.int32),
        pltpu.VMEM((CHUNK,), jnp.float32),
        pltpu.VMEM((CHUNK,), jnp.float32),
        pltpu.VMEM((CHUNK, 16), jnp.float32),
        pltpu.VMEM((CHUNK, 16), jnp.float32),
        pltpu.VMEM_SHARED((N, 16), jnp.float32),
        pltpu.SemaphoreType.DMA,
        pltpu.SemaphoreType.DMA,
    ],
)


def _norm_body(srcf, dstf, ewf, dinv_h, normf, src_v, dst_v, ew_v, nrm_v, dinv_v):
    c = lax.axis_index("c")
    s = lax.axis_index("s")
    w = s * NC + c
    pltpu.sync_copy(srcf.at[w], src_v)
    pltpu.sync_copy(dstf.at[w], dst_v)
    pltpu.sync_copy(ewf.at[w], ew_v)
    pltpu.sync_copy(dinv_h, dinv_v)

    def chunk_body(b, carry):
        sl = pl.ds(b * 16, 16)
        nv = (plsc.load_gather(dinv_v, [src_v[sl]])
              * ew_v[sl]
              * plsc.load_gather(dinv_v, [dst_v[sl]]))
        nrm_v[sl] = nv
        return carry

    lax.fori_loop(0, EPW // 16, chunk_body, 0)
    pltpu.sync_copy(nrm_v, normf.at[w])


_norm_call = pl.kernel(
    _norm_body,
    out_type=jax.ShapeDtypeStruct((NW, EPW), jnp.float32),
    mesh=_MESH,
    compiler_params=_SC_PARAMS,
    scratch_types=[
        pltpu.VMEM((EPW,), jnp.int32),
        pltpu.VMEM((EPW,), jnp.int32),
        pltpu.VMEM((EPW,), jnp.float32),
        pltpu.VMEM((EPW,), jnp.float32),
        pltpu.VMEM((N,), jnp.float32),
    ],
)


def _agg_body(t_h, srcb, dstb, normf, out,
              src0, src1, src2, dst0, dst1, dst2, nrm_v,
              rows0, rows1, rows2, acc,
              sg0, sg1, sg2, ss0, ss1, ss2, sm0, sm1, sm2, sd0, sd1, sd2):
    c = lax.axis_index("c")
    s = lax.axis_index("s")
    w = s * NC + c
    srcs, dsts = (src0, src1, src2), (dst0, dst1, dst2)
    rows = (rows0, rows1, rows2)
    sg, ss, sm, sd = (sg0, sg1, sg2), (ss0, ss1, ss2), (sm0, sm1, sm2), (sd0, sd1, sd2)

    def stage_src(g, i):
        pltpu.async_copy(srcb.at[w, g], srcs[i], sm[i])

    def wait_src(g, i):
        pltpu.make_async_copy(srcb.at[w, g], srcs[i], sm[i]).wait()

    def stage_dst(g, i):
        pltpu.async_copy(dstb.at[w, g], dsts[i], sd[i])

    def wait_dst(g, i):
        pltpu.make_async_copy(dstb.at[w, g], dsts[i], sd[i]).wait()

    def gather(buf_i):
        pltpu.async_copy(t_h.at[srcs[buf_i].at[0]], rows[buf_i], sg[buf_i])

    def wait_gather(buf_i):
        pltpu.make_async_copy(
            t_h.at[srcs[buf_i].at[0]], rows[buf_i], sg[buf_i]).wait()

    def scatter(buf_i):
        pltpu.async_copy(rows[buf_i], acc.at[dsts[buf_i].at[0]], ss[buf_i],
                         add=True)

    def wait_scatter(buf_i):
        pltpu.make_async_copy(
            rows[buf_i], acc.at[dsts[buf_i].at[0]], ss[buf_i]).wait()

    def scale(buf_i, g):
        buf = rows[buf_i]
        gbase = g * ACH

        def body(jj, carry2):
            for u in range(5):
                j = jj * 5 + u
                nrm = plsc.load_gather(
                    nrm_v, [jnp.full((16,), gbase + j, jnp.int32)])
                for cc in range(F // 16):
                    sl = pl.ds(16 * cc, 16)
                    buf[j, sl] = buf[j, sl] * nrm
            return carry2

        lax.fori_loop(0, ACH // 5, body, 0)

    pltpu.sync_copy(normf.at[w], nrm_v)
    for i in range(3):
        pltpu.sync_copy(srcb.at[w, i], srcs[i])
        pltpu.sync_copy(dstb.at[w, i], dsts[i])
    _zero_rows(rows0, 40, F)
    base, ncop = _stripe_info(s)
    _zero_stripe(acc, rows0, base, ncop)
    for i in range(3):
        gather(i)
    plsc.subcore_barrier()

    def tri_body(k, carry):
        c0 = 3 * k

        @pl.when(k > 0)
        def _():
            wait_scatter(2)
            stage_dst(c0 + 2, 2)
            wait_src(c0 + 2, 2)
            gather(2)

        wait_gather(0)
        scale(0, c0)
        stage_src(c0 + 3, 0)

        @pl.when(k > 0)
        def _():
            wait_dst(c0, 0)

        scatter(0)

        wait_gather(1)
        scale(1, c0 + 1)
        stage_src(c0 + 4, 1)

        @pl.when(k > 0)
        def _():
            wait_dst(c0 + 1, 1)

        scatter(1)

        wait_scatter(0)
        stage_dst(c0 + 3, 0)
        wait_src(c0 + 3, 0)
        gather(0)

        wait_gather(2)
        scale(2, c0 + 2)

        @pl.when(k < NTRI - 1)
        def _():
            stage_src(c0 + 5, 2)

        @pl.when(k > 0)
        def _():
            wait_dst(c0 + 2, 2)

        scatter(2)

        wait_scatter(1)
        stage_dst(c0 + 4, 1)
        wait_src(c0 + 4, 1)
        gather(1)

        return carry

    lax.fori_loop(0, NTRI, tri_body, 0)
    wait_scatter(2)
    wait_gather(0)
    scale(0, ANCH - 2)
    wait_dst(ANCH - 2, 0)
    scatter(0)
    wait_gather(1)
    scale(1, ANCH - 1)
    wait_dst(ANCH - 1, 1)
    scatter(1)
    wait_scatter(0)
    wait_scatter(1)
    plsc.subcore_barrier()
    _write_stripe(acc, out, c, base, ncop)


_agg_call = pl.kernel(
    _agg_body,
    out_type=jax.ShapeDtypeStruct((NC, N, F), jnp.float32),
    mesh=_MESH,
    compiler_params=_SC_PARAMS,
    scratch_types=(
        [pltpu.VMEM((1, ACH), jnp.int32)] * 3
        + [pltpu.VMEM((1, ACH), jnp.int32)] * 3
        + [pltpu.VMEM((EPW,), jnp.float32)]
        + [pltpu.VMEM((ACH, F), jnp.float32)] * 3
        + [pltpu.VMEM_SHARED((N, F), jnp.float32)]
        + [pltpu.SemaphoreType.DMA] * 12
    ),
)


def _prep_body(x_ref, nw_ref, nb_ref, degp_ref, t_ref, dinv_ref):
    deg = degp_ref[0, :, 0] + degp_ref[1, :, 0] + 1.0
    dinv_ref[0, 0] = lax.rsqrt(deg)
    t_ref[...] = (jnp.dot(x_ref[...], nw_ref[...],
                          preferred_element_type=jnp.float32) + nb_ref[...])


_prep_call = pl.pallas_call(
    _prep_body,
    grid=(_GRID,),
    in_specs=[
        pl.BlockSpec((_ROWBLK, F), lambda i: (i, 0)),
        pl.BlockSpec((F, F), lambda i: (0, 0)),
        pl.BlockSpec((F,), lambda i: (0,)),
        pl.BlockSpec((NC, _ROWBLK, 16), lambda i: (0, i, 0)),
    ],
    out_specs=[
        pl.BlockSpec((_ROWBLK, F), lambda i: (i, 0)),
        pl.BlockSpec((1, 1, _ROWBLK), lambda i: (i, 0, 0)),
    ],
    out_shape=[
        jax.ShapeDtypeStruct((N, F), jnp.float32),
        jax.ShapeDtypeStruct((_GRID, 1, _ROWBLK), jnp.float32),
    ],
)


def _layer_body(first, p_ref, t_ref, dinv_ref, h_ref, w_ref, b_ref, g_ref,
                bb_ref, ho_ref, to_ref, *lin_refs):
    dinv = dinv_ref[0, 0]
    agg = p_ref[0] + p_ref[1] + t_ref[...] * (dinv * dinv)[:, None]
    conv = jnp.dot(agg, w_ref[...], preferred_element_type=jnp.float32) + b_ref[...]
    h = conv if first else h_ref[...] + conv
    if ho_ref is not None:
        ho_ref[...] = h
    mu = jnp.mean(h, axis=-1, keepdims=True)
    var = jnp.mean((h - mu) ** 2, axis=-1, keepdims=True)
    t_next = jnp.maximum(
        (h - mu) * lax.rsqrt(var + 1e-5) * g_ref[...] + bb_ref[...], 0.0)
    if lin_refs:
        lw_ref, lb_ref = lin_refs
        t_next = (jnp.dot(t_next, lw_ref[...],
                          preferred_element_type=jnp.float32) + lb_ref[...])
    to_ref[...] = t_next


def _make_layer_call(first, last):
    def body(*refs):
        if last:
            (p, t, dinv, h, w, b, g, bb, lw, lb, to) = refs
            _layer_body(first, p, t, dinv, h, w, b, g, bb, None, to, lw, lb)
        else:
            (p, t, dinv, h, w, b, g, bb, ho, to) = refs
            _layer_body(first, p, t, dinv, h, w, b, g, bb, ho, to)

    odim = ODIM if last else F
    in_specs = [
        pl.BlockSpec((NC, _ROWBLK, F), lambda i: (0, i, 0)),
        pl.BlockSpec((_ROWBLK, F), lambda i: (i, 0)),
        pl.BlockSpec((1, 1, _ROWBLK), lambda i: (i, 0, 0)),
        pl.BlockSpec((_ROWBLK, F), lambda i: (i, 0)),
        pl.BlockSpec((F, F), lambda i: (0, 0)),
        pl.BlockSpec((F,), lambda i: (0,)),
        pl.BlockSpec((F,), lambda i: (0,)),
        pl.BlockSpec((F,), lambda i: (0,)),
    ]
    if last:
        in_specs += [
            pl.BlockSpec((F, ODIM), lambda i: (0, 0)),
            pl.BlockSpec((ODIM,), lambda i: (0,)),
        ]
        out_specs = pl.BlockSpec((_ROWBLK, ODIM), lambda i: (i, 0))
        out_shape = jax.ShapeDtypeStruct((N, ODIM), jnp.float32)
    else:
        out_specs = [
            pl.BlockSpec((_ROWBLK, F), lambda i: (i, 0)),
            pl.BlockSpec((_ROWBLK, F), lambda i: (i, 0)),
        ]
        out_shape = [
            jax.ShapeDtypeStruct((N, F), jnp.float32),
            jax.ShapeDtypeStruct((N, F), jnp.float32),
        ]
    return pl.pallas_call(
        body,
        grid=(_GRID,),
        in_specs=in_specs,
        out_specs=out_specs,
        out_shape=out_shape,
    )


_layer_first = _make_layer_call(True, False)
_layer_rest = _make_layer_call(False, False)
_layer_last = _make_layer_call(False, True)


def kernel(x, edge_attr, node_W, node_b, conv_W, conv_b, ln_g, ln_b, lin_W,
           lin_b, edge_index):
    srcf = edge_index[0].reshape(NW, EPW)
    dstf = edge_index[1].reshape(NW, EPW)
    srcb = edge_index[0].reshape(NW, ANCH, 1, ACH)
    dstb = edge_index[1].reshape(NW, ANCH, 1, ACH)
    dstb_deg = edge_index[1].reshape(NW, NCHUNK, CHUNK)
    ewf = edge_attr.reshape(NW, EPW)
    ewb = edge_attr.reshape(NW, NCHUNK, CHUNK)

    degp = _deg_call(dstb_deg, ewb)
    t, dinv = _prep_call(x, node_W, node_b, degp)
    normb = _norm_call(srcf, dstf, ewf, dinv.reshape(N))

    h = t
    for i in range(NLAYERS):
        p = _agg_call(t, srcb, dstb, normb)
        gi = ln_g[i + 1] if i + 1 < NLAYERS else ln_g[0]
        bi = ln_b[i + 1] if i + 1 < NLAYERS else ln_b[0]
        if i == 0:
            h, t = _layer_first(p, t, dinv, h, conv_W[i], conv_b[i], gi, bi)
        elif i < NLAYERS - 1:
            h, t = _layer_rest(p, t, dinv, h, conv_W[i], conv_b[i], gi, bi)
        else:
            return _layer_last(p, t, dinv, h, conv_W[i], conv_b[i], gi, bi,
                               lin_W, lin_b)
